# Initial kernel scaffold; baseline (speedup 1.0000x reference)
import jax
import jax.numpy as jnp
from jax.experimental import pallas as pl


def kernel(g_indices, g_values, emb_weight):
    raise NotImplementedError("write your pallas kernel here")


# speedup vs baseline: 4.0977x; 4.0977x over previous
import functools

import jax
import jax.numpy as jnp
from jax import lax
from jax.experimental import pallas as pl
from jax.experimental.pallas import tpu as pltpu
from jax.experimental.pallas import tpu_sc as plsc

N = 10000
E = 320000
D = 128

NC = 2
NS = 16
NW = NC * NS
EPW = E // NW
C = 80
CHUNKS = EPW // C
INIT_BLK = 200
NBLK = N // INIT_BLK
INIT_STEPS = -(-NBLK // NS)


def _sc_spmm(rows_hbm, cols_hbm, gv_hbm, emb_hbm, part_hbm,
             acc, cols_v, rows_v, gv_v, msg_v, init_v, sem):
    cid = lax.axis_index("c")
    sid = lax.axis_index("s")
    wid = cid * NS + sid

    def init_body(k, _):
        b = sid + k * NS

        @pl.when(b < NBLK)
        def _():
            row0 = b * INIT_BLK
            pltpu.sync_copy(emb_hbm.at[pl.ds(row0, INIT_BLK)], init_v)

            def scale_row(i, _):
                for d in range(D // 16):
                    sl = pl.ds(d * 16, 16)
                    init_v[i, sl] = init_v[i, sl] * 0.125
                return 0

            lax.fori_loop(0, INIT_BLK, scale_row, 0)
            pltpu.sync_copy(init_v, acc.at[pl.ds(row0, INIT_BLK)])

        return 0

    lax.fori_loop(0, INIT_STEPS, init_body, 0)
    plsc.subcore_barrier()

    def chunk_body(ci, _):
        base = wid * EPW + ci * C
        pltpu.sync_copy(cols_hbm.at[pl.ds(base, C)], cols_v)
        pltpu.sync_copy(rows_hbm.at[pl.ds(base, C)], rows_v)
        pltpu.sync_copy(gv_hbm.at[pl.ds(base, C)], gv_v)
        pltpu.async_copy(emb_hbm.at[cols_v], msg_v, sem).wait()

        def scale_group(g, _):
            gvv = gv_v[pl.ds(g * 16, 16)] * 0.75
            for j in range(16):
                i = g * 16 + j
                gvb = jnp.full((16,), gvv[j], jnp.float32)
                for d in range(D // 16):
                    sl = pl.ds(d * 16, 16)
                    msg_v[i, sl] = msg_v[i, sl] * gvb
            return 0

        lax.fori_loop(0, C // 16, scale_group, 0)
        pltpu.sync_copy(msg_v, acc.at[rows_v], add=True)
        return 0

    lax.fori_loop(0, CHUNKS, chunk_body, 0)
    plsc.subcore_barrier()

    def out_body(k, _):
        b = sid + k * NS

        @pl.when(b < NBLK)
        def _():
            row0 = b * INIT_BLK
            pltpu.sync_copy(acc.at[pl.ds(row0, INIT_BLK)], init_v)
            pltpu.sync_copy(init_v, part_hbm.at[cid, pl.ds(row0, INIT_BLK)])

        return 0

    lax.fori_loop(0, INIT_STEPS, out_body, 0)


def _combine_body(p_ref, o_ref):
    o_ref[...] = p_ref[0] + p_ref[1]


def kernel(g_indices, g_values, emb_weight):
    rows = g_indices[0].astype(jnp.int32)
    cols = g_indices[1].astype(jnp.int32)
    gv = g_values.astype(jnp.float32)
    emb = emb_weight.astype(jnp.float32)

    mesh = plsc.VectorSubcoreMesh(core_axis_name="c", subcore_axis_name="s")
    spmm = functools.partial(
        pl.kernel,
        out_type=jax.ShapeDtypeStruct((NC, N, D), jnp.float32),
        mesh=mesh,
        scratch_types=[
            pltpu.VMEM_SHARED((N, D), jnp.float32),
            pltpu.VMEM((C,), jnp.int32),
            pltpu.VMEM((C,), jnp.int32),
            pltpu.VMEM((C,), jnp.float32),
            pltpu.VMEM((C, D), jnp.float32),
            pltpu.VMEM((INIT_BLK, D), jnp.float32),
            pltpu.SemaphoreType.DMA,
        ],
    )(_sc_spmm)
    partials = spmm(rows, cols, gv, emb)

    blk = 1000
    out = pl.pallas_call(
        _combine_body,
        grid=(N // blk,),
        in_specs=[pl.BlockSpec((NC, blk, D), lambda i: (0, i, 0))],
        out_specs=pl.BlockSpec((blk, D), lambda i: (i, 0)),
        out_shape=jax.ShapeDtypeStruct((N, D), jnp.float32),
    )(partials)
    return out


# speedup vs baseline: 10.0204x; 2.4454x over previous
import functools

import jax
import jax.numpy as jnp
from jax import lax
from jax.experimental import pallas as pl
from jax.experimental.pallas import tpu as pltpu
from jax.experimental.pallas import tpu_sc as plsc

N = 10000
E = 320000
D = 128

NC = 2
NS = 16
NW = NC * NS
EPW = E // NW
C = 80
CHUNKS = EPW // C
NB = 4
OBLK = 200
NOB = N // OBLK


def _sc_spmm(edata_hbm, emb_hbm, part_hbm, acc,
             msg, ebuf, gsem, esem, ssem):
    cid = lax.axis_index("c")
    sid = lax.axis_index("s")
    wid = cid * NS + sid

    def zero_row(i, _):
        for d in range(D // 16):
            msg[0][i, pl.ds(d * 16, 16)] = jnp.zeros((16,), jnp.float32)
        return 0

    lax.fori_loop(0, C, zero_row, 0)

    def zero_blk(k, _):
        b = sid + k * NS

        @pl.when(b < CHUNKS)
        def _():
            pltpu.sync_copy(msg[0], acc.at[pl.ds(b * C, C)])

        return 0

    lax.fori_loop(0, -(-CHUNKS // NS), zero_blk, 0)
    plsc.subcore_barrier()

    def start_edata(ci, b):
        pltpu.async_copy(edata_hbm.at[wid, ci], ebuf[b], esem[b])

    def wait_edata(b):
        pltpu.make_async_copy(edata_hbm.at[0, 0], ebuf[b], esem[b]).wait()

    def start_gather(ci, b):
        pltpu.async_copy(emb_hbm.at[ebuf[b].at[1]], msg[b], gsem[b])

    def wait_gather(b):
        pltpu.make_async_copy(emb_hbm.at[pl.ds(0, C)], msg[b], gsem[b]).wait()

    def start_scatter(ci, b):
        pltpu.async_copy(msg[b], acc.at[ebuf[b].at[0]], ssem[b], add=True)

    def wait_scatter(b):
        pltpu.make_async_copy(emb_hbm.at[pl.ds(0, C)], msg[b], ssem[b]).wait()

    def scale(b):
        def grp(g, _):
            bits = ebuf[b][2, pl.ds(g * 16, 16)]
            gvv = lax.bitcast_convert_type(bits, jnp.float32) * 0.75
            for j in range(16):
                gvb = jnp.full((16,), gvv[j], jnp.float32)
                i = g * 16 + j
                for d in range(D // 16):
                    sl = pl.ds(d * 16, 16)
                    msg[b][i, sl] = msg[b][i, sl] * gvb
            return 0

        lax.fori_loop(0, C // 16, grp, 0)

    def step(ci, b):
        @pl.when(ci >= 2)
        def _():
            wait_scatter((b + 2) % NB)

        @pl.when(ci + 2 < CHUNKS)
        def _():
            start_edata(ci + 2, (b + 2) % NB)

        wait_edata((b + 1) % NB)
        start_gather(ci + 1, (b + 1) % NB)
        wait_gather(b)
        scale(b)
        start_scatter(ci, b)

    start_edata(0, 0)
    start_edata(1, 1)
    wait_edata(0)
    start_gather(0, 0)

    def quad(k, _):
        for j in range(NB):
            step(NB * k + j, j)
        return 0

    lax.fori_loop(0, (CHUNKS - 1) // NB, quad, 0)
    wait_scatter(2)
    wait_gather(0)
    scale(0)
    start_scatter(CHUNKS - 1, 0)
    wait_scatter(3)
    wait_scatter(0)
    plsc.subcore_barrier()

    def out_body(k, _):
        b = sid + k * NS

        @pl.when(b < NOB)
        def _():
            row0 = b * OBLK
            pltpu.sync_copy(acc.at[pl.ds(row0, OBLK)],
                            part_hbm.at[cid, pl.ds(row0, OBLK)])

        return 0

    lax.fori_loop(0, -(-NOB // NS), out_body, 0)


def _combine_body(p_ref, e_ref, o_ref):
    o_ref[...] = p_ref[0] + p_ref[1] + 0.25 * e_ref[...]


def kernel(g_indices, g_values, emb_weight):
    rows = g_indices[0].astype(jnp.int32).reshape(NW, CHUNKS, C)
    cols = g_indices[1].astype(jnp.int32).reshape(NW, CHUNKS, C)
    gvb = jax.lax.bitcast_convert_type(
        g_values.astype(jnp.float32), jnp.int32).reshape(NW, CHUNKS, C)
    edata = jnp.stack([rows, cols, gvb], axis=2)
    emb = emb_weight.astype(jnp.float32)

    mesh = plsc.VectorSubcoreMesh(core_axis_name="c", subcore_axis_name="s")
    spmm = functools.partial(
        pl.kernel,
        out_type=jax.ShapeDtypeStruct((NC, N, D), jnp.float32),
        mesh=mesh,
        scratch_types=[
            pltpu.VMEM_SHARED((N, D), jnp.float32),
            [pltpu.VMEM((C, D), jnp.float32) for _ in range(NB)],
            [pltpu.VMEM((3, C), jnp.int32) for _ in range(NB)],
            [pltpu.SemaphoreType.DMA for _ in range(NB)],
            [pltpu.SemaphoreType.DMA for _ in range(NB)],
            [pltpu.SemaphoreType.DMA for _ in range(NB)],
        ],
    )(_sc_spmm)
    partials = spmm(edata, emb)

    blk = 1000
    out = pl.pallas_call(
        _combine_body,
        grid=(N // blk,),
        in_specs=[
            pl.BlockSpec((NC, blk, D), lambda i: (0, i, 0)),
            pl.BlockSpec((blk, D), lambda i: (i, 0)),
        ],
        out_specs=pl.BlockSpec((blk, D), lambda i: (i, 0)),
        out_shape=jax.ShapeDtypeStruct((N, D), jnp.float32),
    )(partials, emb)
    return out


# speedup vs baseline: 10.6000x; 1.0578x over previous
import functools

import jax
import jax.numpy as jnp
from jax import lax
from jax.experimental import pallas as pl
from jax.experimental.pallas import tpu as pltpu
from jax.experimental.pallas import tpu_sc as plsc

N = 10000
E = 320000
D = 128

NC = 2
NS = 16
NW = NC * NS
EPW = E // NW
C = 80
CHUNKS = EPW // C
NB = 4
OBLK = 200
NOB = N // OBLK


def _sc_spmm(edr_hbm, edc_hbm, emb_hbm, part_hbm, acc,
             msg, ebr, ebc, gsem, rsem, csem, ssem):
    cid = lax.axis_index("c")
    sid = lax.axis_index("s")
    wid = cid * NS + sid

    def zero_row(i, _):
        for d in range(D // 16):
            msg[0][i, pl.ds(d * 16, 16)] = jnp.zeros((16,), jnp.float32)
        return 0

    lax.fori_loop(0, C, zero_row, 0)

    def zero_blk(k, _):
        b = sid + k * NS

        @pl.when(b < CHUNKS)
        def _():
            pltpu.sync_copy(msg[0], acc.at[pl.ds(b * C, C)])

        return 0

    lax.fori_loop(0, -(-CHUNKS // NS), zero_blk, 0)
    plsc.subcore_barrier()

    def start_edr(ci, b):
        pltpu.async_copy(edr_hbm.at[wid, ci], ebr[b], rsem[b])

    def wait_edr(b):
        pltpu.make_async_copy(edr_hbm.at[0, 0], ebr[b], rsem[b]).wait()

    def start_edc(ci, b):
        pltpu.async_copy(edc_hbm.at[wid, ci], ebc[b], csem[b])

    def wait_edc(b):
        pltpu.make_async_copy(edc_hbm.at[0, 0], ebc[b], csem[b]).wait()

    def start_gather(ci, b):
        pltpu.async_copy(emb_hbm.at[ebc[b].at[0]], msg[b], gsem[b])

    def wait_gather(b):
        pltpu.make_async_copy(emb_hbm.at[pl.ds(0, C)], msg[b], gsem[b]).wait()

    def start_scatter(ci, b):
        pltpu.async_copy(msg[b], acc.at[ebr[b].at[0]], ssem[b], add=True)

    def wait_scatter(b):
        pltpu.make_async_copy(emb_hbm.at[pl.ds(0, C)], msg[b], ssem[b]).wait()

    def scale(b):
        def grp(g, _):
            bits = ebc[b][1, pl.ds(g * 16, 16)]
            gvv = lax.bitcast_convert_type(bits, jnp.float32) * 0.75
            for j in range(16):
                gvb = jnp.full((16,), gvv[j], jnp.float32)
                i = g * 16 + j
                for d in range(D // 16):
                    sl = pl.ds(d * 16, 16)
                    msg[b][i, sl] = msg[b][i, sl] * gvb
            return 0

        lax.fori_loop(0, C // 16, grp, 0)

    def step(ci, b):
        @pl.when(ci >= 2)
        def _():
            wait_scatter((b + 2) % NB)

        @pl.when(ci + 2 < CHUNKS)
        def _():
            start_edr(ci + 2, (b + 2) % NB)

        @pl.when(ci + 3 < CHUNKS)
        def _():
            start_edc(ci + 3, (b + 3) % NB)

        @pl.when(ci + 2 < CHUNKS)
        def _():
            wait_edc((b + 2) % NB)
            start_gather(ci + 2, (b + 2) % NB)

        wait_gather(b)
        scale(b)
        wait_edr(b)
        start_scatter(ci, b)

    start_edc(0, 0)
    start_edc(1, 1)
    start_edc(2, 2)
    start_edr(0, 0)
    start_edr(1, 1)
    wait_edc(0)
    start_gather(0, 0)
    wait_edc(1)
    start_gather(1, 1)

    def quad(k, _):
        for j in range(NB):
            step(NB * k + j, j)
        return 0

    lax.fori_loop(0, (CHUNKS - 1) // NB, quad, 0)
    wait_scatter(2)
    wait_gather(0)
    scale(0)
    wait_edr(0)
    start_scatter(CHUNKS - 1, 0)
    wait_scatter(3)
    wait_scatter(0)
    plsc.subcore_barrier()

    def out_body(k, _):
        b = sid + k * NS

        @pl.when(b < NOB)
        def _():
            row0 = b * OBLK
            pltpu.sync_copy(acc.at[pl.ds(row0, OBLK)],
                            part_hbm.at[cid, pl.ds(row0, OBLK)])

        return 0

    lax.fori_loop(0, -(-NOB // NS), out_body, 0)


def _combine_body(p_ref, e_ref, o_ref):
    o_ref[...] = p_ref[0] + p_ref[1] + 0.25 * e_ref[...]


def kernel(g_indices, g_values, emb_weight):
    rows = g_indices[0].astype(jnp.int32).reshape(NW, CHUNKS, 1, C)
    cols = g_indices[1].astype(jnp.int32).reshape(NW, CHUNKS, C)
    gvb = jax.lax.bitcast_convert_type(
        g_values.astype(jnp.float32), jnp.int32).reshape(NW, CHUNKS, C)
    edc = jnp.stack([cols, gvb], axis=2)
    emb = emb_weight.astype(jnp.float32)

    mesh = plsc.VectorSubcoreMesh(core_axis_name="c", subcore_axis_name="s")
    spmm = functools.partial(
        pl.kernel,
        out_type=jax.ShapeDtypeStruct((NC, N, D), jnp.float32),
        mesh=mesh,
        scratch_types=[
            pltpu.VMEM_SHARED((N, D), jnp.float32),
            [pltpu.VMEM((C, D), jnp.float32) for _ in range(NB)],
            [pltpu.VMEM((1, C), jnp.int32) for _ in range(NB)],
            [pltpu.VMEM((2, C), jnp.int32) for _ in range(NB)],
            [pltpu.SemaphoreType.DMA for _ in range(NB)],
            [pltpu.SemaphoreType.DMA for _ in range(NB)],
            [pltpu.SemaphoreType.DMA for _ in range(NB)],
            [pltpu.SemaphoreType.DMA for _ in range(NB)],
        ],
    )(_sc_spmm)
    partials = spmm(rows, edc, emb)

    blk = 1000
    out = pl.pallas_call(
        _combine_body,
        grid=(N // blk,),
        in_specs=[
            pl.BlockSpec((NC, blk, D), lambda i: (0, i, 0)),
            pl.BlockSpec((blk, D), lambda i: (i, 0)),
        ],
        out_specs=pl.BlockSpec((blk, D), lambda i: (i, 0)),
        out_shape=jax.ShapeDtypeStruct((N, D), jnp.float32),
    )(partials, emb)
    return out
